# Initial kernel scaffold; baseline (speedup 1.0000x reference)
import jax
import jax.numpy as jnp
from jax.experimental import pallas as pl


def kernel(atoms_embed, che_nbrs_fea, che_nbrs_idx, vdw_nbrs_fea, vdw_nbrs_idx, num_atoms, params):
    raise NotImplementedError("write your pallas kernel here")


# speedup vs baseline: 1.3961x; 1.3961x over previous
import functools

import jax
import jax.numpy as jnp
import numpy as np
from jax import lax
from jax.experimental import pallas as pl
from jax.experimental.pallas import tpu as pltpu
from jax.experimental.pallas import tpu_sc as plsc

N = 10000
M = 32
B = 100
NODE_IN = 13
K_RBF = 20
H = 128
CHE_CUTOFF = 8.0
VDW_CUTOFF = 12.0
E = N * M


def _softplus(x):
    return jnp.maximum(x, 0.0) + jnp.log(1.0 + jnp.exp(-jnp.abs(x)))


def _sigmoid(x):
    s = 1.0 / (1.0 + jnp.exp(-jnp.abs(x)))
    return jnp.where(x >= 0, s, 1.0 - s)


_EMBED_BN = 400


def _embed_body(a_ref, w_ref, b_ref, o_ref):
    o_ref[:] = (
        jnp.dot(a_ref[:], w_ref[:], preferred_element_type=jnp.float32) + b_ref[:]
    )


def _embed(atoms_embed, W, b):
    return pl.pallas_call(
        _embed_body,
        grid=(N // _EMBED_BN,),
        in_specs=[
            pl.BlockSpec((_EMBED_BN, NODE_IN), lambda i: (i, 0)),
            pl.BlockSpec((NODE_IN, H), lambda i: (0, 0)),
            pl.BlockSpec((1, H), lambda i: (0, 0)),
        ],
        out_specs=pl.BlockSpec((_EMBED_BN, H), lambda i: (i, 0)),
        out_shape=jax.ShapeDtypeStruct((N, H), jnp.float32),
    )(atoms_embed, W, b.reshape(1, H))


_RBF_BE = 1000


def _rbf_body(d_ref, o_ref, *, cutoff):
    d = d_ref[:]
    k = lax.broadcasted_iota(jnp.int32, (1, K_RBF), 1).astype(jnp.float32) + 1.0
    r = jnp.where(d < cutoff, jnp.sin(d * k * (np.pi / cutoff)) / d, 0.0)
    w = 0.5 * (jnp.cos(d * (np.pi / cutoff)) + 1.0)
    o_ref[:] = r * w


def _rbf(fea, cutoff):
    return pl.pallas_call(
        functools.partial(_rbf_body, cutoff=cutoff),
        grid=(E // _RBF_BE,),
        in_specs=[pl.BlockSpec((_RBF_BE, 1), lambda i: (i, 0))],
        out_specs=pl.BlockSpec((_RBF_BE, K_RBF), lambda i: (i, 0)),
        out_shape=jax.ShapeDtypeStruct((E, K_RBF), jnp.float32),
    )(fea.reshape(E, 1))


_GNW = 32
_GCH = 80
_G_TOTAL = 2 * E
_G_PERW = _G_TOTAL // _GNW
_G_NCH = _G_PERW // _GCH

@functools.cache
def _gather_sc():
    mesh = plsc.VectorSubcoreMesh(core_axis_name="c", subcore_axis_name="s")

    @functools.partial(
        pl.kernel,
        mesh=mesh,
        out_type=jax.ShapeDtypeStruct((_G_TOTAL, H), jnp.float32),
        scratch_types=[
            pltpu.VMEM((_G_NCH, _GCH), jnp.int32),
            pltpu.VMEM((_GCH, H), jnp.float32),
            pltpu.SemaphoreType.DMA,
        ],
    )
    def gather_k(table_hbm, idx_hbm, out_hbm, idx_v, rows_v, sem):
        wid = lax.axis_index("s") * 2 + lax.axis_index("c")
        base = wid * _G_PERW
        pltpu.sync_copy(idx_hbm.at[wid], idx_v)

        def body(j, carry):
            pltpu.async_copy(table_hbm.at[idx_v.at[j]], rows_v, sem).wait()
            pltpu.sync_copy(rows_v, out_hbm.at[pl.ds(base + j * _GCH, _GCH)])
            return carry

        lax.fori_loop(0, _G_NCH, body, 0)

    return gather_k


def _gather(nodes, idx3):
    return _gather_sc()(nodes, idx3)


_CONV_BN = 80
_CONV_GRID = N // _CONV_BN
_CONV_BE = _CONV_BN * M


def _conv_body(n_ref, rc_ref, rv_ref, gc_ref, gv_ref,
               cWa, cWe, cWc, cbg, vWa, vWe, vWc, vbg, o_ref):
    nodes = n_ref[:]

    def branch(r_ref, g_ref, Wa, We, Wc, bg):
        A = jnp.dot(nodes, Wa[:], preferred_element_type=jnp.float32) + bg[:]
        Ee = jnp.dot(r_ref[:], We[:], preferred_element_type=jnp.float32)
        Nf = jnp.dot(g_ref[:], Wc[:], preferred_element_type=jnp.float32)
        g = (Ee + Nf).reshape(_CONV_BN, M, 2 * H) + A[:, None, :]
        s = _sigmoid(g[:, :, :H]) * _softplus(g[:, :, H:])
        return jnp.sum(s, axis=1)

    aggr = branch(rc_ref, gc_ref, cWa, cWe, cWc, cbg)
    aggr += branch(rv_ref, gv_ref, vWa, vWe, vWc, vbg)
    o_ref[:] = _softplus(nodes + aggr)


def _conv(nodes, rbf_che, rbf_vdw, gathered, wts):
    nblk = _CONV_GRID
    full = lambda shape: pl.BlockSpec(shape, lambda i: (0, 0))
    return pl.pallas_call(
        _conv_body,
        grid=(nblk,),
        in_specs=[
            pl.BlockSpec((_CONV_BN, H), lambda i: (i, 0)),
            pl.BlockSpec((_CONV_BE, K_RBF), lambda i: (i, 0)),
            pl.BlockSpec((_CONV_BE, K_RBF), lambda i: (i, 0)),
            pl.BlockSpec((_CONV_BE, H), lambda i: (i, 0)),
            pl.BlockSpec((_CONV_BE, H), lambda i: (i + nblk, 0)),
            full((H, 2 * H)), full((K_RBF, 2 * H)), full((H, 2 * H)),
            full((1, 2 * H)),
            full((H, 2 * H)), full((K_RBF, 2 * H)), full((H, 2 * H)),
            full((1, 2 * H)),
        ],
        out_specs=pl.BlockSpec((_CONV_BN, H), lambda i: (i, 0)),
        out_shape=jax.ShapeDtypeStruct((N, H), jnp.float32),
    )(nodes, rbf_che, rbf_vdw, gathered, gathered, *wts)


def _head_body(p_ref, wf_ref, bf_ref, wo_ref, bo_ref, o_ref):
    x = _softplus(p_ref[:])
    x = jnp.dot(x, wf_ref[:], preferred_element_type=jnp.float32) + bf_ref[:]
    x = _softplus(x)
    o_ref[:] = (
        jnp.dot(x, wo_ref[:], preferred_element_type=jnp.float32) + bo_ref[:]
    )


def _head(pooled, W_fc, b_fc, W_out, b_out):
    return pl.pallas_call(
        _head_body,
        out_shape=jax.ShapeDtypeStruct((B, 1), jnp.float32),
    )(pooled, W_fc, b_fc.reshape(1, H), W_out, b_out.reshape(1, 1))


def kernel(atoms_embed, che_nbrs_fea, che_nbrs_idx, vdw_nbrs_fea,
           vdw_nbrs_idx, num_atoms, params):
    nodes = _embed(atoms_embed, params['W_embed'], params['b_embed'])
    rbf_che = _rbf(che_nbrs_fea, CHE_CUTOFF)
    rbf_vdw = _rbf(vdw_nbrs_fea, VDW_CUTOFF)

    idx3 = jnp.concatenate(
        [che_nbrs_idx.reshape(E), vdw_nbrs_idx.reshape(E)]
    ).reshape(_GNW, _G_NCH, _GCH)

    for c in params['convs']:
        wts = []
        for p in ('che', 'vdw'):
            gW, gb = c[f'{p}_gW'], c[f'{p}_gb']
            fW, fb = c[f'{p}_fW'], c[f'{p}_fb']
            wts += [gW[:H], fW @ gW[H:2 * H], gW[2 * H:],
                    (gb + fb @ gW[H:2 * H]).reshape(1, 2 * H)]
        gathered = _gather(nodes, idx3)
        nodes = _conv(nodes, rbf_che, rbf_vdw, gathered, wts)

    pooled = nodes[:B] / num_atoms.astype(jnp.float32)[:, None]
    out = _head(pooled, params['W_fc'], params['b_fc'],
                params['W_out'], params['b_out'])
    return out.reshape(B)
